# initial kernel scaffold (unmeasured)
import jax
import jax.numpy as jnp
from jax import lax
from jax.experimental import pallas as pl
from jax.experimental.pallas import tpu as pltpu

N_DEV = 32
BLK = 128


def _a2a(x_shard):
    k_glob, m_blk = x_shard.shape

    def body(x_ref, out_ref, send_sems, recv_sems):
        my = lax.axis_index("i")

        out_ref[:, pl.ds(my * BLK, BLK)] = x_ref[pl.ds(my * BLK, BLK), :]

        for d in range(1, N_DEV):
            dst = lax.rem(my + d, N_DEV)
            rdma = pltpu.make_async_remote_copy(
                src_ref=x_ref.at[pl.ds(dst * BLK, BLK), :],
                dst_ref=out_ref.at[:, pl.ds(my * BLK, BLK)],
                send_sem=send_sems.at[d],
                recv_sem=recv_sems.at[my],
                device_id=(dst,),
                device_id_type=pl.DeviceIdType.MESH,
            )
            rdma.start()

        for d in range(1, N_DEV):
            src = lax.rem(my + N_DEV - d, N_DEV)
            send_wait = pltpu.make_async_remote_copy(
                src_ref=x_ref.at[pl.ds(0, BLK), :],
                dst_ref=out_ref.at[:, pl.ds(0, BLK)],
                send_sem=send_sems.at[d],
                recv_sem=recv_sems.at[0],
                device_id=(my,),
                device_id_type=pl.DeviceIdType.MESH,
            )
            send_wait.wait_send()
            recv_wait = pltpu.make_async_remote_copy(
                src_ref=x_ref.at[pl.ds(0, BLK), :],
                dst_ref=out_ref.at[:, pl.ds(src * BLK, BLK)],
                send_sem=send_sems.at[0],
                recv_sem=recv_sems.at[src],
                device_id=(my,),
                device_id_type=pl.DeviceIdType.MESH,
            )
            recv_wait.wait_recv()

    return pl.pallas_call(
        body,
        out_shape=jax.ShapeDtypeStruct((m_blk, k_glob), x_shard.dtype),
        in_specs=[pl.BlockSpec(memory_space=pltpu.VMEM)],
        out_specs=pl.BlockSpec(memory_space=pltpu.VMEM),
        scratch_shapes=[
            pltpu.SemaphoreType.DMA((N_DEV,)),
            pltpu.SemaphoreType.DMA((N_DEV,)),
        ],
        compiler_params=pltpu.CompilerParams(collective_id=0),
    )(x_shard)


def _gemm(x_row, w_mat):
    m, k = x_row.shape
    _, n = w_mat.shape
    bn = 512
    grid = (n // bn,)

    def body(x_ref, w_ref, o_ref):
        o_ref[...] = jnp.dot(
            x_ref[...], w_ref[...], preferred_element_type=jnp.float32
        )

    return pl.pallas_call(
        body,
        grid=grid,
        in_specs=[
            pl.BlockSpec((m, k), lambda j: (0, 0)),
            pl.BlockSpec((k, bn), lambda j: (0, j)),
        ],
        out_specs=pl.BlockSpec((m, bn), lambda j: (0, j)),
        out_shape=jax.ShapeDtypeStruct((m, n), jnp.float32),
    )(x_row, w_mat)


def kernel(x, w_mat):
    x_row = _a2a(x)
    return _gemm(x_row, w_mat)


# baseline (device time: 82278 ns/iter reference)
import jax
import jax.numpy as jnp
from jax import lax
from jax.experimental import pallas as pl
from jax.experimental.pallas import tpu as pltpu

N_DEV = 32
BLK = 128


def _a2a(x_shard):
    k_glob, m_blk = x_shard.shape

    def body(x_ref, out_ref, send_sems, recv_sems):
        my = lax.axis_index("i")

        out_ref[:, pl.ds(my * BLK, BLK)] = x_ref[pl.ds(my * BLK, BLK), :]

        for d in range(1, N_DEV):
            dst = lax.rem(my + d, N_DEV)
            rdma = pltpu.make_async_remote_copy(
                src_ref=x_ref.at[pl.ds(dst * BLK, BLK), :],
                dst_ref=out_ref.at[:, pl.ds(my * BLK, BLK)],
                send_sem=send_sems.at[d],
                recv_sem=recv_sems.at[my],
                device_id=(dst,),
                device_id_type=pl.DeviceIdType.MESH,
            )
            rdma.start()

        for d in range(1, N_DEV):
            src = lax.rem(my + N_DEV - d, N_DEV)
            send_wait = pltpu.make_async_remote_copy(
                src_ref=x_ref.at[pl.ds(0, BLK), :],
                dst_ref=out_ref.at[:, pl.ds(0, BLK)],
                send_sem=send_sems.at[d],
                recv_sem=recv_sems.at[0],
                device_id=(my,),
                device_id_type=pl.DeviceIdType.MESH,
            )
            send_wait.wait_send()
            recv_wait = pltpu.make_async_remote_copy(
                src_ref=x_ref.at[pl.ds(0, BLK), :],
                dst_ref=out_ref.at[:, pl.ds(src * BLK, BLK)],
                send_sem=send_sems.at[0],
                recv_sem=recv_sems.at[src],
                device_id=(my,),
                device_id_type=pl.DeviceIdType.MESH,
            )
            recv_wait.wait_recv()

    return pl.pallas_call(
        body,
        out_shape=jax.ShapeDtypeStruct((m_blk, k_glob), x_shard.dtype),
        in_specs=[pl.BlockSpec(memory_space=pltpu.VMEM)],
        out_specs=pl.BlockSpec(memory_space=pltpu.VMEM),
        scratch_shapes=[
            pltpu.SemaphoreType.DMA((N_DEV,)),
            pltpu.SemaphoreType.DMA((N_DEV,)),
        ],
    )(x_shard)


def _gemm(x_row, w_mat):
    m, k = x_row.shape
    _, n = w_mat.shape
    bn = 512
    grid = (n // bn,)

    def body(x_ref, w_ref, o_ref):
        o_ref[...] = jnp.dot(
            x_ref[...], w_ref[...], preferred_element_type=jnp.float32
        )

    return pl.pallas_call(
        body,
        grid=grid,
        in_specs=[
            pl.BlockSpec((m, k), lambda j: (0, 0)),
            pl.BlockSpec((k, bn), lambda j: (0, j)),
        ],
        out_specs=pl.BlockSpec((m, bn), lambda j: (0, j)),
        out_shape=jax.ShapeDtypeStruct((m, n), jnp.float32),
    )(x_row, w_mat)


def kernel(x, w_mat):
    x_row = _a2a(x)
    return _gemm(x_row, w_mat)
